# baseline (speedup 1.0000x reference)
import functools

import jax
import jax.numpy as jnp
from jax import lax
from jax.experimental import pallas as pl
from jax.experimental.pallas import tpu as pltpu
from jax.experimental.pallas import tpu_sc as plsc

_NE = 8192
_D = 256
_NT = 8192
_COMMIT = 0.25

_TB = 256
_A = _TB // 32


def _fused_body(x_ref, w_ref, dist_ref, enc_ref, mi_ref, loss_ref, ppl_ref,
                cnt_s, w2_s, acc_s):
    i = pl.program_id(0)
    ni = pl.num_programs(0)
    x = jnp.transpose(x_ref[...])
    w = w_ref[...]

    @pl.when(i == 0)
    def _():
        w2_s[...] = jnp.sum(w * w, axis=1)

    x2 = jnp.sum(x * x, axis=1, keepdims=True)
    mm = lax.dot_general(x, w, (((1,), (1,)), ((), ())),
                         preferred_element_type=jnp.float32)
    dist = (x2 + w2_s[...][None, :]) - 2.0 * mm
    dist_ref[...] = dist.reshape(_A, 512, 512)

    tmin = jnp.min(dist, axis=1)
    cols = lax.broadcasted_iota(jnp.int32, (_TB, _NE), 1)
    targ = jnp.min(jnp.where(dist == tmin[:, None], cols, _NE),
                   axis=1)
    mi_ref[...] = targ

    onehot = (targ[:, None] == cols).astype(jnp.float32)
    enc_ref[...] = onehot.reshape(_A, 512, 512)
    colsum = jnp.sum(onehot, axis=0)

    @pl.when(i == 0)
    def _():
        cnt_s[...] = colsum
        acc_s[0] = jnp.sum(tmin)

    @pl.when(i > 0)
    def _():
        cnt_s[...] = cnt_s[...] + colsum
        acc_s[0] = acc_s[0] + jnp.sum(tmin)

    @pl.when(i == ni - 1)
    def _():
        loss_ref[0, 0] = (1.0 + _COMMIT) * acc_s[0] * (1.0 / (_NT * _D))
        p = cnt_s[...] * (1.0 / _NT)
        ppl_ref[0, 0] = jnp.exp(-jnp.sum(p * jnp.log(p + 1e-10)))


def _fused_call(flat_x, weight):
    return pl.pallas_call(
        _fused_body,
        grid=(_NT // _TB,),
        in_specs=[
            pl.BlockSpec((_D, _TB), lambda i: (0, i)),
            pl.BlockSpec((_NE, _D), lambda i: (0, 0)),
        ],
        out_specs=[
            pl.BlockSpec((_A, 512, 512), lambda i: (i, 0, 0)),
            pl.BlockSpec((_A, 512, 512), lambda i: (i, 0, 0)),
            pl.BlockSpec((_TB,), lambda i: (i,)),
            pl.BlockSpec(memory_space=pltpu.SMEM),
            pl.BlockSpec(memory_space=pltpu.SMEM),
        ],
        out_shape=[
            jax.ShapeDtypeStruct((256, 512, 512), jnp.float32),
            jax.ShapeDtypeStruct((256, 512, 512), jnp.float32),
            jax.ShapeDtypeStruct((_NT,), jnp.int32),
            jax.ShapeDtypeStruct((1, 1), jnp.float32),
            jax.ShapeDtypeStruct((1, 1), jnp.float32),
        ],
        scratch_shapes=[
            pltpu.VMEM((_NE,), jnp.float32),
            pltpu.VMEM((_NE,), jnp.float32),
            pltpu.SMEM((2,), jnp.float32),
        ],
    )(flat_x, weight)


def _sc_gather(weight, minidx):
    info = plsc.get_sparse_core_info()
    nw = info.num_cores * info.num_subcores
    bpw = _NT // nw
    nchunks = bpw // 128
    idx2 = minidx.reshape(_NT // 128, 128)
    mesh = plsc.VectorSubcoreMesh(core_axis_name="c", subcore_axis_name="s")

    @functools.partial(
        pl.kernel, mesh=mesh,
        out_type=jax.ShapeDtypeStruct((_NT, _D), jnp.float32),
        scratch_types=[
            pltpu.VMEM((nchunks, 128), jnp.int32),
            pltpu.VMEM((bpw, _D), jnp.float32),
            pltpu.SemaphoreType.DMA,
        ],
    )
    def k(w_hbm, idx_hbm, out_hbm, idx_v, rows_v, sem):
        wid = lax.axis_index("s") * info.num_cores + lax.axis_index("c")
        pltpu.sync_copy(idx_hbm.at[pl.ds(wid * nchunks, nchunks)], idx_v)
        for r in range(nchunks):
            pltpu.async_copy(w_hbm.at[idx_v.at[r]],
                             rows_v.at[pl.ds(r * 128, 128)], sem).wait()
        pltpu.sync_copy(rows_v, out_hbm.at[pl.ds(wid * bpw, bpw)])

    return k(weight, idx2)


def kernel(inputs, weight):
    flat_xt = inputs.reshape(_D, _NT)
    dist3, enc3, minidx, loss, ppl = _fused_call(flat_xt, weight)
    quant = _sc_gather(weight, minidx)
    out_q = jnp.transpose(quant.reshape(16, 512, _D), (2, 0, 1))
    return (loss.reshape(()),
            out_q,
            ppl.reshape(()),
            enc3,
            dist3,
            minidx[:, None])


# speedup vs baseline: 1.0008x; 1.0008x over previous
import functools

import jax
import jax.numpy as jnp
from jax import lax
from jax.experimental import pallas as pl
from jax.experimental.pallas import tpu as pltpu
from jax.experimental.pallas import tpu_sc as plsc

_NE = 8192
_D = 256
_NT = 8192
_COMMIT = 0.25

_TB = 256
_A = _TB // 32


def _fused_body(x_ref, w_ref, dist_ref, enc_ref, mi_ref, loss_ref, ppl_ref,
                cnt_s, w2_s, acc_s):
    i = pl.program_id(0)
    ni = pl.num_programs(0)
    x = x_ref[...]
    w = w_ref[...]

    @pl.when(i == 0)
    def _():
        w2_s[...] = jnp.sum(w * w, axis=1)

    x2 = jnp.sum(x * x, axis=1, keepdims=True)
    mm = lax.dot_general(x, w, (((1,), (1,)), ((), ())),
                         preferred_element_type=jnp.float32)
    dist = (x2 + w2_s[...][None, :]) - 2.0 * mm
    dist_ref[...] = dist.reshape(_A, 512, 512)

    tmin = jnp.min(dist, axis=1)
    cols = lax.broadcasted_iota(jnp.int32, (_TB, _NE), 1)
    targ = jnp.min(jnp.where(dist == tmin[:, None], cols, _NE),
                   axis=1)
    mi_ref[...] = targ

    onehot = (targ[:, None] == cols).astype(jnp.float32)
    enc_ref[...] = onehot.reshape(_A, 512, 512)
    colsum = jnp.sum(onehot, axis=0)

    @pl.when(i == 0)
    def _():
        cnt_s[...] = colsum
        acc_s[0] = jnp.sum(tmin)

    @pl.when(i > 0)
    def _():
        cnt_s[...] = cnt_s[...] + colsum
        acc_s[0] = acc_s[0] + jnp.sum(tmin)

    @pl.when(i == ni - 1)
    def _():
        loss_ref[0, 0] = (1.0 + _COMMIT) * acc_s[0] * (1.0 / (_NT * _D))
        p = cnt_s[...] * (1.0 / _NT)
        ppl_ref[0, 0] = jnp.exp(-jnp.sum(p * jnp.log(p + 1e-10)))


def _fused_call(flat_x, weight):
    return pl.pallas_call(
        _fused_body,
        grid=(_NT // _TB,),
        in_specs=[
            pl.BlockSpec((_TB, _D), lambda i: (i, 0)),
            pl.BlockSpec((_NE, _D), lambda i: (0, 0)),
        ],
        out_specs=[
            pl.BlockSpec((_A, 512, 512), lambda i: (i, 0, 0)),
            pl.BlockSpec((_A, 512, 512), lambda i: (i, 0, 0)),
            pl.BlockSpec((_TB,), lambda i: (i,)),
            pl.BlockSpec(memory_space=pltpu.SMEM),
            pl.BlockSpec(memory_space=pltpu.SMEM),
        ],
        out_shape=[
            jax.ShapeDtypeStruct((256, 512, 512), jnp.float32),
            jax.ShapeDtypeStruct((256, 512, 512), jnp.float32),
            jax.ShapeDtypeStruct((_NT,), jnp.int32),
            jax.ShapeDtypeStruct((1, 1), jnp.float32),
            jax.ShapeDtypeStruct((1, 1), jnp.float32),
        ],
        scratch_shapes=[
            pltpu.VMEM((_NE,), jnp.float32),
            pltpu.VMEM((_NE,), jnp.float32),
            pltpu.SMEM((2,), jnp.float32),
        ],
    )(flat_x, weight)


def _sc_gather(weight, minidx):
    info = plsc.get_sparse_core_info()
    nw = info.num_cores * info.num_subcores
    bpw = _NT // nw
    nchunks = bpw // 128
    idx2 = minidx.reshape(_NT // 128, 128)
    mesh = plsc.VectorSubcoreMesh(core_axis_name="c", subcore_axis_name="s")

    @functools.partial(
        pl.kernel, mesh=mesh,
        out_type=jax.ShapeDtypeStruct((_NT, _D), jnp.float32),
        scratch_types=[
            pltpu.VMEM((nchunks, 128), jnp.int32),
            pltpu.VMEM((bpw, _D), jnp.float32),
            pltpu.SemaphoreType.DMA,
        ],
    )
    def k(w_hbm, idx_hbm, out_hbm, idx_v, rows_v, sem):
        wid = lax.axis_index("s") * info.num_cores + lax.axis_index("c")
        pltpu.sync_copy(idx_hbm.at[pl.ds(wid * nchunks, nchunks)], idx_v)
        for r in range(nchunks):
            pltpu.async_copy(w_hbm.at[idx_v.at[r]],
                             rows_v.at[pl.ds(r * 128, 128)], sem).wait()
        pltpu.sync_copy(rows_v, out_hbm.at[pl.ds(wid * bpw, bpw)])

    return k(weight, idx2)


def kernel(inputs, weight):
    flat_x = jnp.transpose(inputs, (1, 2, 0)).reshape(_NT, _D)
    dist3, enc3, minidx, loss, ppl = _fused_call(flat_x, weight)
    quant = _sc_gather(weight, minidx)
    out_q = jnp.transpose(quant.reshape(16, 512, _D), (2, 0, 1))
    return (loss.reshape(()),
            out_q,
            ppl.reshape(()),
            enc3,
            dist3,
            minidx[:, None])
